# baseline (speedup 1.0000x reference)
import functools

import jax
import jax.numpy as jnp
from jax import lax
from jax.experimental import pallas as pl
from jax.experimental.pallas import tpu as pltpu
from jax.experimental.pallas import tpu_sc as plsc

N = 10000
E = 320000
D = 128
HID = 256
R = 3
B = 64

NC = 2
NS = 16
NW = NC * NS
EP = E // NW
C = 128
EPP = 10240
NCHUNK = EPP // C
RB = 80
NBLK = N // RB
BLK_ITERS = (NBLK + NS - 1) // NS

_mesh = plsc.VectorSubcoreMesh(core_axis_name="c", subcore_axis_name="s",
                               num_cores=NC, num_subcores=NS)


@functools.partial(
    pl.kernel,
    out_type=jax.ShapeDtypeStruct((NC, N, D), jnp.float32),
    mesh=_mesh,
    scratch_types=[
        pltpu.VMEM((EPP,), jnp.int32),
        pltpu.VMEM((EPP,), jnp.float32),
        pltpu.VMEM((NCHUNK, C), jnp.int32),
        pltpu.VMEM((C, D), jnp.float32),
        pltpu.VMEM_SHARED((N, D), jnp.float32),
        pltpu.SemaphoreType.DMA,
    ],
)
def _sc_edge_kernel(yflat_hbm, eg_hbm, ew_hbm, ed_hbm, out_hbm,
                    gidx_v, w_v, dst_v, rows_v, acc, sem):
    c = lax.axis_index("c")
    s = lax.axis_index("s")
    wid = c * NS + s

    z16 = jnp.zeros((16,), jnp.float32)

    def zero_body(r, carry):
        for k in range(D // 16):
            rows_v[r, pl.ds(k * 16, 16)] = z16
        return carry

    lax.fori_loop(0, C, zero_body, 0)
    for t in range(BLK_ITERS):
        blk = s + t * NS

        @pl.when(blk < NBLK)
        def _():
            start = pl.multiple_of(blk * RB, 8)
            pltpu.sync_copy(rows_v.at[pl.ds(0, RB)], acc.at[pl.ds(start, RB)])

    pltpu.sync_copy(eg_hbm.at[wid], gidx_v)
    pltpu.sync_copy(ew_hbm.at[wid], w_v)
    pltpu.sync_copy(ed_hbm.at[wid], dst_v)

    def prep_body(i, carry):
        sl = pl.ds(i * 16, 16)
        g16 = gidx_v[sl]
        is1 = (g16 >= N) & (g16 < 2 * N)
        w16 = w_v[sl]
        w_v[sl] = jnp.where(is1, (w16 + 1.0) * 0.5, jnp.float32(1.0))
        return carry

    lax.fori_loop(0, EPP // 16, prep_body, 0)
    plsc.subcore_barrier()

    def chunk_body(i, carry):
        pltpu.async_copy(yflat_hbm.at[gidx_v.at[pl.ds(i * C, C)]],
                         rows_v, sem).wait()

        def scale_body(g, carry2):
            cf16 = w_v[pl.ds(i * C + g * 16, 16)]
            for j in range(16):
                row = g * 16 + j
                cf = lax.gather(
                    cf16, jnp.full((16, 1), j, jnp.int32),
                    dimension_numbers=lax.GatherDimensionNumbers(
                        offset_dims=(), collapsed_slice_dims=(0,),
                        start_index_map=(0,)),
                    slice_sizes=(1,),
                    mode=lax.GatherScatterMode.PROMISE_IN_BOUNDS)
                for k in range(D // 16):
                    sl = pl.ds(k * 16, 16)
                    rows_v[row, sl] = rows_v[row, sl] * cf
            return carry2

        lax.fori_loop(0, C // 16, scale_body, 0)
        pltpu.sync_copy(rows_v, acc.at[dst_v.at[i]], add=True)
        return carry

    lax.fori_loop(0, NCHUNK, chunk_body, 0)
    plsc.subcore_barrier()

    for t in range(BLK_ITERS):
        blk = s + t * NS

        @pl.when(blk < NBLK)
        def _():
            start = pl.multiple_of(blk * RB, 8)
            sl = pl.ds(start, RB)
            pltpu.sync_copy(acc.at[sl], out_hbm.at[c, sl])


def _mm_body(x_ref, w_ref, o_ref):
    o_ref[0] = lax.dot_general(x_ref[...], w_ref[0],
                               (((1,), (1,)), ((), ())),
                               preferred_element_type=jnp.float32)


_mm_call = pl.pallas_call(
    _mm_body,
    grid=(R,),
    in_specs=[
        pl.BlockSpec((N, D), lambda r: (0, 0)),
        pl.BlockSpec((1, D, D), lambda r: (r, 0, 0)),
    ],
    out_specs=pl.BlockSpec((1, N, D), lambda r: (r, 0, 0)),
    out_shape=jax.ShapeDtypeStruct((R, N, D), jnp.float32),
)


def _combine_body(part_ref, x_ref, gamma_ref, beta_ref, o_ref):
    h = part_ref[0] + part_ref[1] + x_ref[...]
    mean = jnp.mean(h, axis=0, keepdims=True)
    var = jnp.mean(h * h, axis=0, keepdims=True) - mean * mean
    h = gamma_ref[...] * (h - mean) / jnp.sqrt(var + 1e-5) + beta_ref[...]
    o_ref[...] = jnp.maximum(h, 0.0)


_combine_call = pl.pallas_call(
    _combine_body,
    out_shape=jax.ShapeDtypeStruct((N, D), jnp.float32),
)


def _head_body(h_ref, c_ref, w1_ref, b1_ref, g1_ref, be1_ref,
               w2_ref, b2_ref, w3_ref, b3_ref, o_ref):
    idx = c_ref[...]
    cols = lax.broadcasted_iota(jnp.int32, (B, N), 1)
    onehot = (cols == idx).astype(jnp.float32)
    emb = lax.dot_general(onehot, h_ref[...], (((1,), (0,)), ((), ())),
                          preferred_element_type=jnp.float32)
    z = lax.dot_general(emb, w1_ref[...], (((1,), (1,)), ((), ())),
                        preferred_element_type=jnp.float32) + b1_ref[...]
    mean = jnp.mean(z, axis=0, keepdims=True)
    var = jnp.mean(z * z, axis=0, keepdims=True) - mean * mean
    z = g1_ref[...] * (z - mean) / jnp.sqrt(var + 1e-5) + be1_ref[...]
    z = jnp.maximum(z, 0.0)
    z = lax.dot_general(z, w2_ref[...], (((1,), (1,)), ((), ())),
                        preferred_element_type=jnp.float32) + b2_ref[...]
    z = jnp.maximum(z, 0.0)
    o_ref[...] = lax.dot_general(z, w3_ref[...], (((1,), (1,)), ((), ())),
                                 preferred_element_type=jnp.float32) + b3_ref[...]


_head_call = pl.pallas_call(
    _head_body,
    out_shape=jax.ShapeDtypeStruct((B, 8), jnp.float32),
)


def kernel(node_features, edge_index, edge_type, edge_weight,
           claim_node_indices, W_gnn, bn_gamma, bn_beta,
           W1, b1, g1, be1, W2, b2, W3, b3):
    pad = EPP - EP
    gidx = (edge_type * N + edge_index[0]).reshape(NW, EP)
    eg = jnp.pad(gidx, ((0, 0), (0, pad)), constant_values=N)
    ew = jnp.pad(edge_weight.reshape(NW, EP), ((0, 0), (0, pad)),
                 constant_values=-1.0)
    ed = jnp.pad(edge_index[1].reshape(NW, EP),
                 ((0, 0), (0, pad))).reshape(NW, NCHUNK, C)

    h = node_features
    for i in range(2):
        yflat = _mm_call(h, W_gnn[i]).reshape(R * N, D)
        part = _sc_edge_kernel(yflat, eg, ew, ed)
        h = _combine_call(part, h, bn_gamma[i].reshape(1, D),
                          bn_beta[i].reshape(1, D))

    w3p = jnp.zeros((8, HID // 2), jnp.float32).at[:2].set(W3)
    b3p = jnp.zeros((1, 8), jnp.float32).at[0, :2].set(b3)
    logits = _head_call(h, claim_node_indices.reshape(B, 1), W1,
                        b1.reshape(1, HID), g1.reshape(1, HID),
                        be1.reshape(1, HID), W2, b2.reshape(1, HID // 2),
                        w3p, b3p)
    return logits[:, :2]


# speedup vs baseline: 2.0370x; 2.0370x over previous
import functools

import jax
import jax.numpy as jnp
from jax import lax
from jax.experimental import pallas as pl
from jax.experimental.pallas import tpu as pltpu
from jax.experimental.pallas import tpu_sc as plsc

N = 10000
E = 320000
D = 128
HID = 256
R = 3
B = 64

NC = 2
NS = 16
NW = NC * NS
EP = E // NW
C = 128
EPP = 10240
NCHUNK = EPP // C
RB = 80
NBLK = N // RB
BLK_ITERS = (NBLK + NS - 1) // NS

_mesh = plsc.VectorSubcoreMesh(core_axis_name="c", subcore_axis_name="s",
                               num_cores=NC, num_subcores=NS)


@functools.partial(
    pl.kernel,
    out_type=jax.ShapeDtypeStruct((NC, N, D), jnp.float32),
    mesh=_mesh,
    scratch_types=[
        pltpu.VMEM((EPP,), jnp.int32),
        pltpu.VMEM((EPP,), jnp.float32),
        pltpu.VMEM((NCHUNK, C), jnp.int32),
        pltpu.VMEM((C, D), jnp.float32),
        pltpu.VMEM_SHARED((N, D), jnp.float32),
        pltpu.SemaphoreType.DMA,
    ],
)
def _sc_edge_kernel(yflat_hbm, eg_hbm, ew_hbm, ed_hbm, out_hbm,
                    gidx_v, w_v, dst_v, rows_v, acc, sem):
    c = lax.axis_index("c")
    s = lax.axis_index("s")
    wid = c * NS + s

    z16 = jnp.zeros((16,), jnp.float32)

    def zero_body(r, carry):
        for k in range(D // 16):
            rows_v[r, pl.ds(k * 16, 16)] = z16
        return carry

    lax.fori_loop(0, C, zero_body, 0)
    for t in range(BLK_ITERS):
        blk = s + t * NS

        @pl.when(blk < NBLK)
        def _():
            start = pl.multiple_of(blk * RB, 8)
            pltpu.sync_copy(rows_v.at[pl.ds(0, RB)], acc.at[pl.ds(start, RB)])

    pltpu.sync_copy(eg_hbm.at[wid], gidx_v)
    pltpu.sync_copy(ew_hbm.at[wid], w_v)
    pltpu.sync_copy(ed_hbm.at[wid], dst_v)

    def prep_body(i, carry):
        sl = pl.ds(i * 16, 16)
        g16 = gidx_v[sl]
        is1 = (g16 >= N) & (g16 < 2 * N)
        w16 = w_v[sl]
        w_v[sl] = jnp.where(is1, (w16 + 1.0) * 0.5, jnp.float32(1.0))
        return carry

    lax.fori_loop(0, EPP // 16, prep_body, 0)
    plsc.subcore_barrier()

    def chunk_body(i, carry):
        pltpu.async_copy(yflat_hbm.at[gidx_v.at[pl.ds(i * C, C)]],
                         rows_v, sem).wait()

        def scale_body(g, carry2):
            cf16 = w_v[pl.ds(i * C + g * 16, 16)]
            for j in range(16):
                row = g * 16 + j
                cf = lax.gather(
                    cf16, jnp.full((16, 1), j, jnp.int32),
                    dimension_numbers=lax.GatherDimensionNumbers(
                        offset_dims=(), collapsed_slice_dims=(0,),
                        start_index_map=(0,)),
                    slice_sizes=(1,),
                    mode=lax.GatherScatterMode.PROMISE_IN_BOUNDS)
                for k in range(D // 16):
                    sl = pl.ds(k * 16, 16)
                    rows_v[row, sl] = rows_v[row, sl] * cf
            return carry2

        lax.fori_loop(0, C // 16, scale_body, 0)
        pltpu.sync_copy(rows_v, acc.at[dst_v.at[i]], add=True)
        return carry

    lax.fori_loop(0, NCHUNK, chunk_body, 0)
    plsc.subcore_barrier()

    for t in range(BLK_ITERS):
        blk = s + t * NS

        @pl.when(blk < NBLK)
        def _():
            start = pl.multiple_of(blk * RB, 8)
            sl = pl.ds(start, RB)
            pltpu.sync_copy(acc.at[sl], out_hbm.at[c, sl])


def _mm_body(x_ref, w_ref, o_ref):
    o_ref[0] = lax.dot_general(x_ref[...], w_ref[0],
                               (((1,), (1,)), ((), ())),
                               preferred_element_type=jnp.float32)


_mm_call = pl.pallas_call(
    _mm_body,
    grid=(R,),
    in_specs=[
        pl.BlockSpec((N, D), lambda r: (0, 0)),
        pl.BlockSpec((1, D, D), lambda r: (r, 0, 0)),
    ],
    out_specs=pl.BlockSpec((1, N, D), lambda r: (r, 0, 0)),
    out_shape=jax.ShapeDtypeStruct((R, N, D), jnp.float32),
)


def _combine_body(part_ref, x_ref, gamma_ref, beta_ref, o_ref):
    h = part_ref[0] + part_ref[1] + x_ref[...]
    mean = jnp.mean(h, axis=0, keepdims=True)
    var = jnp.mean(h * h, axis=0, keepdims=True) - mean * mean
    h = gamma_ref[...] * (h - mean) / jnp.sqrt(var + 1e-5) + beta_ref[...]
    o_ref[...] = jnp.maximum(h, 0.0)


_combine_call = pl.pallas_call(
    _combine_body,
    out_shape=jax.ShapeDtypeStruct((N, D), jnp.float32),
)


def _head_body(h_ref, c_ref, w1_ref, b1_ref, g1_ref, be1_ref,
               w2_ref, b2_ref, w3_ref, b3_ref, o_ref):
    idx = c_ref[...]
    cols = lax.broadcasted_iota(jnp.int32, (B, N), 1)
    onehot = (cols == idx).astype(jnp.float32)
    emb = lax.dot_general(onehot, h_ref[...], (((1,), (0,)), ((), ())),
                          preferred_element_type=jnp.float32)
    z = lax.dot_general(emb, w1_ref[...], (((1,), (1,)), ((), ())),
                        preferred_element_type=jnp.float32) + b1_ref[...]
    mean = jnp.mean(z, axis=0, keepdims=True)
    var = jnp.mean(z * z, axis=0, keepdims=True) - mean * mean
    z = g1_ref[...] * (z - mean) / jnp.sqrt(var + 1e-5) + be1_ref[...]
    z = jnp.maximum(z, 0.0)
    z = lax.dot_general(z, w2_ref[...], (((1,), (1,)), ((), ())),
                        preferred_element_type=jnp.float32) + b2_ref[...]
    z = jnp.maximum(z, 0.0)
    o_ref[...] = lax.dot_general(z, w3_ref[...], (((1,), (1,)), ((), ())),
                                 preferred_element_type=jnp.float32) + b3_ref[...]


_head_call = pl.pallas_call(
    _head_body,
    out_shape=jax.ShapeDtypeStruct((B, 8), jnp.float32),
)


def kernel(node_features, edge_index, edge_type, edge_weight,
           claim_node_indices, W_gnn, bn_gamma, bn_beta,
           W1, b1, g1, be1, W2, b2, W3, b3):
    pad = EPP - EP
    spread = (jnp.arange(NW * pad, dtype=jnp.int32) * 97) % N
    gidx = (edge_type * N + edge_index[0]).reshape(NW, EP)
    eg = jnp.concatenate(
        [gidx, N + spread.reshape(NW, pad)], axis=1)
    ew = jnp.pad(edge_weight.reshape(NW, EP), ((0, 0), (0, pad)),
                 constant_values=-1.0)
    ed = jnp.concatenate(
        [edge_index[1].reshape(NW, EP), spread.reshape(NW, pad)],
        axis=1).reshape(NW, NCHUNK, C)

    h = node_features
    for i in range(2):
        yflat = _mm_call(h, W_gnn[i]).reshape(R * N, D)
        part = _sc_edge_kernel(yflat, eg, ew, ed)
        h = _combine_call(part, h, bn_gamma[i].reshape(1, D),
                          bn_beta[i].reshape(1, D))

    w3p = jnp.zeros((8, HID // 2), jnp.float32).at[:2].set(W3)
    b3p = jnp.zeros((1, 8), jnp.float32).at[0, :2].set(b3)
    logits = _head_call(h, claim_node_indices.reshape(B, 1), W1,
                        b1.reshape(1, HID), g1.reshape(1, HID),
                        be1.reshape(1, HID), W2, b2.reshape(1, HID // 2),
                        w3p, b3p)
    return logits[:, :2]


# speedup vs baseline: 3.0251x; 1.4851x over previous
import functools

import jax
import jax.numpy as jnp
from jax import lax
from jax.experimental import pallas as pl
from jax.experimental.pallas import tpu as pltpu
from jax.experimental.pallas import tpu_sc as plsc

N = 10000
E = 320000
D = 128
HID = 256
R = 3
B = 64

NC = 2
NS = 16
NW = NC * NS
EP = E // NW
C = 128
EPP = 10240
NCHUNK = EPP // C
NPH = 2
EPH = EPP // NPH
CPH = EPH // C
RB = 80
NBLK = N // RB
BLK_ITERS = (NBLK + NS - 1) // NS

_mesh = plsc.VectorSubcoreMesh(core_axis_name="c", subcore_axis_name="s",
                               num_cores=NC, num_subcores=NS)


@functools.partial(
    pl.kernel,
    out_type=jax.ShapeDtypeStruct((NC, N, D), jnp.float32),
    mesh=_mesh,
    scratch_types=[
        pltpu.VMEM((EPH,), jnp.int32),
        pltpu.VMEM((EPH,), jnp.float32),
        pltpu.VMEM((CPH, C), jnp.int32),
        pltpu.VMEM((C, D), jnp.float32),
        pltpu.VMEM((C, D), jnp.float32),
        pltpu.VMEM_SHARED((N, D), jnp.float32),
        pltpu.SemaphoreType.DMA,
    ],
)
def _sc_edge_kernel(yflat_hbm, eg_hbm, ew_hbm, ed_hbm, out_hbm,
                    gidx_v, w_v, dst_v, rows_a, rows_b, acc, sem):
    c = lax.axis_index("c")
    s = lax.axis_index("s")
    wid = c * NS + s

    z16 = jnp.zeros((16,), jnp.float32)

    def zero_body(r, carry):
        for k in range(D // 16):
            rows_a[r, pl.ds(k * 16, 16)] = z16
        return carry

    lax.fori_loop(0, C, zero_body, 0)
    for t in range(BLK_ITERS):
        blk = s + t * NS

        @pl.when(blk < NBLK)
        def _():
            start = pl.multiple_of(blk * RB, 8)
            pltpu.sync_copy(rows_a.at[pl.ds(0, RB)], acc.at[pl.ds(start, RB)])

    plsc.subcore_barrier()

    for ph in range(NPH):
        pltpu.sync_copy(eg_hbm.at[wid, ph], gidx_v)
        pltpu.sync_copy(ew_hbm.at[wid, ph], w_v)
        pltpu.sync_copy(ed_hbm.at[wid, ph], dst_v)

        def prep_body(i, carry):
            sl = pl.ds(i * 16, 16)
            g16 = gidx_v[sl]
            is1 = (g16 >= N) & (g16 < 2 * N)
            w16 = w_v[sl]
            w_v[sl] = jnp.where(is1, (w16 + 1.0) * 0.5, jnp.float32(1.0))
            return carry

        lax.fori_loop(0, EPH // 16, prep_body, 0)

        pltpu.async_copy(yflat_hbm.at[gidx_v.at[pl.ds(0, C)]],
                         rows_a, sem)

        def do_chunk(i, rows_v):
            def scale_body(g, carry2):
                cf16 = w_v[pl.ds(i * C + g * 16, 16)]
                for j in range(16):
                    row = g * 16 + j
                    cf = lax.gather(
                        cf16, jnp.full((16, 1), j, jnp.int32),
                        dimension_numbers=lax.GatherDimensionNumbers(
                            offset_dims=(), collapsed_slice_dims=(0,),
                            start_index_map=(0,)),
                        slice_sizes=(1,),
                        mode=lax.GatherScatterMode.PROMISE_IN_BOUNDS)
                    for k in range(D // 16):
                        sl = pl.ds(k * 16, 16)
                        rows_v[row, sl] = rows_v[row, sl] * cf
                return carry2

            lax.fori_loop(0, C // 16, scale_body, 0)
            pltpu.sync_copy(rows_v, acc.at[dst_v.at[i]], add=True)

        def pair_body(t, carry):
            ia = 2 * t
            ib = 2 * t + 1
            pltpu.make_async_copy(yflat_hbm.at[gidx_v.at[pl.ds(0, C)]],
                                  rows_a, sem).wait()
            pltpu.async_copy(yflat_hbm.at[gidx_v.at[pl.ds(ib * C, C)]],
                             rows_b, sem)
            do_chunk(ia, rows_a)
            pltpu.make_async_copy(yflat_hbm.at[gidx_v.at[pl.ds(0, C)]],
                                  rows_b, sem).wait()

            @pl.when(ib < CPH - 1)
            def _():
                pltpu.async_copy(
                    yflat_hbm.at[gidx_v.at[pl.ds((ib + 1) * C, C)]],
                    rows_a, sem)

            do_chunk(ib, rows_b)
            return carry

        lax.fori_loop(0, CPH // 2, pair_body, 0)
    plsc.subcore_barrier()


    for t in range(BLK_ITERS):
        blk = s + t * NS

        @pl.when(blk < NBLK)
        def _():
            start = pl.multiple_of(blk * RB, 8)
            sl = pl.ds(start, RB)
            pltpu.sync_copy(acc.at[sl], out_hbm.at[c, sl])


def _mm_body(x_ref, w_ref, o_ref):
    o_ref[0] = lax.dot_general(x_ref[...], w_ref[0],
                               (((1,), (1,)), ((), ())),
                               preferred_element_type=jnp.float32)


_mm_call = pl.pallas_call(
    _mm_body,
    grid=(R,),
    in_specs=[
        pl.BlockSpec((N, D), lambda r: (0, 0)),
        pl.BlockSpec((1, D, D), lambda r: (r, 0, 0)),
    ],
    out_specs=pl.BlockSpec((1, N, D), lambda r: (r, 0, 0)),
    out_shape=jax.ShapeDtypeStruct((R, N, D), jnp.float32),
)


def _combine_body(part_ref, x_ref, gamma_ref, beta_ref, o_ref):
    h = part_ref[0] + part_ref[1] + x_ref[...]
    mean = jnp.mean(h, axis=0, keepdims=True)
    var = jnp.mean(h * h, axis=0, keepdims=True) - mean * mean
    h = gamma_ref[...] * (h - mean) / jnp.sqrt(var + 1e-5) + beta_ref[...]
    o_ref[...] = jnp.maximum(h, 0.0)


_combine_call = pl.pallas_call(
    _combine_body,
    out_shape=jax.ShapeDtypeStruct((N, D), jnp.float32),
)


def _head_body(h_ref, c_ref, w1_ref, b1_ref, g1_ref, be1_ref,
               w2_ref, b2_ref, w3_ref, b3_ref, o_ref):
    idx = c_ref[...]
    cols = lax.broadcasted_iota(jnp.int32, (B, N), 1)
    onehot = (cols == idx).astype(jnp.float32)
    emb = lax.dot_general(onehot, h_ref[...], (((1,), (0,)), ((), ())),
                          preferred_element_type=jnp.float32)
    z = lax.dot_general(emb, w1_ref[...], (((1,), (1,)), ((), ())),
                        preferred_element_type=jnp.float32) + b1_ref[...]
    mean = jnp.mean(z, axis=0, keepdims=True)
    var = jnp.mean(z * z, axis=0, keepdims=True) - mean * mean
    z = g1_ref[...] * (z - mean) / jnp.sqrt(var + 1e-5) + be1_ref[...]
    z = jnp.maximum(z, 0.0)
    z = lax.dot_general(z, w2_ref[...], (((1,), (1,)), ((), ())),
                        preferred_element_type=jnp.float32) + b2_ref[...]
    z = jnp.maximum(z, 0.0)
    o_ref[...] = lax.dot_general(z, w3_ref[...], (((1,), (1,)), ((), ())),
                                 preferred_element_type=jnp.float32) + b3_ref[...]


_head_call = pl.pallas_call(
    _head_body,
    out_shape=jax.ShapeDtypeStruct((B, 8), jnp.float32),
)


def kernel(node_features, edge_index, edge_type, edge_weight,
           claim_node_indices, W_gnn, bn_gamma, bn_beta,
           W1, b1, g1, be1, W2, b2, W3, b3):
    pad = EPP - EP
    spread = (jnp.arange(NW * pad, dtype=jnp.int32) * 97) % N
    gidx = (edge_type * N + edge_index[0]).reshape(NW, EP)
    eg = jnp.concatenate(
        [gidx, N + spread.reshape(NW, pad)], axis=1).reshape(NW, NPH, EPH)
    ew = jnp.pad(edge_weight.reshape(NW, EP), ((0, 0), (0, pad)),
                 constant_values=-1.0).reshape(NW, NPH, EPH)
    ed = jnp.concatenate(
        [edge_index[1].reshape(NW, EP), spread.reshape(NW, pad)],
        axis=1).reshape(NW, NPH, CPH, C)

    h = node_features
    for i in range(2):
        yflat = _mm_call(h, W_gnn[i]).reshape(R * N, D)
        part = _sc_edge_kernel(yflat, eg, ew, ed)
        h = _combine_call(part, h, bn_gamma[i].reshape(1, D),
                          bn_beta[i].reshape(1, D))

    w3p = jnp.zeros((8, HID // 2), jnp.float32).at[:2].set(W3)
    b3p = jnp.zeros((1, 8), jnp.float32).at[0, :2].set(b3)
    logits = _head_call(h, claim_node_indices.reshape(B, 1), W1,
                        b1.reshape(1, HID), g1.reshape(1, HID),
                        be1.reshape(1, HID), W2, b2.reshape(1, HID // 2),
                        w3p, b3p)
    return logits[:, :2]
